# baseline (speedup 1.0000x reference)
import numpy as np
import jax
import jax.numpy as jnp
from jax import lax
from jax.experimental import pallas as pl
from jax.experimental.pallas import tpu as pltpu
from jax.experimental.pallas import tpu_sc as plsc

P = 16
S = 2
C = 3
H = W = 512
OH = OW = (H - P) // S + 1
L = OH * OW
F = C * P * P
D = W // 2
NT = C * P


def _proj_matrix():
    p = np.zeros((2 * NT, F), np.float32)
    for t in range(NT):
        for v in range(2):
            for m in range(8):
                p[48 * v + t, 16 * t + 2 * m + v] = 1.0
    return jnp.asarray(p, jnp.bfloat16)


def _window16(ref, i, img_c):
    q = jnp.minimum(i // 4, (H - 24) // 8)
    base = pl.multiple_of(8 * q, 8)
    rows24 = ref[img_c, pl.ds(base, 24), :]
    r = S * i - 8 * q
    return jax.lax.switch(
        r // 2, [lambda k=k: rows24[2 * k:2 * k + P, :] for k in range(5)]
    )


def _rows48(ref, i):
    return jnp.concatenate([_window16(ref, i, c) for c in range(C)], axis=0)


def _make_block(xe, xo, pm, i):
    eo96 = jnp.concatenate([_rows48(xe, i), _rows48(xo, i)], axis=0)
    tr = jnp.swapaxes(eo96, 0, 1)
    hi = tr.astype(jnp.bfloat16)
    lo = (tr - hi.astype(jnp.float32)).astype(jnp.bfloat16)
    dn = (((1,), (0,)), ((), ()))
    rep = (jax.lax.dot_general(hi, pm, dn, preferred_element_type=jnp.float32)
           + jax.lax.dot_general(lo, pm, dn, preferred_element_type=jnp.float32))
    lm = (jax.lax.broadcasted_iota(jnp.int32, (1, F), 1) // 2) % 8
    acc = jnp.where(lm == 0, rep[0:OW, :], 0.0)
    for m in range(1, 8):
        acc = jnp.where(lm == m, rep[m:m + OW, :], acc)
    return acc


OHB = 8
NB = (OH + OHB - 1) // OHB


def _tc_body(xe, xo, p_ref, o_ref):
    b = pl.program_id(0)
    pm = p_ref[...]
    for k in range(OHB):
        i = jnp.minimum(OHB * b + k, OH - 1)
        o_ref[pl.ds(OW * k, OW), :] = _make_block(xe, xo, pm, i)


def _tc_unfold(image):
    xe, xo = image[0, :, :, 0::2], image[0, :, :, 1::2]
    proj = _proj_matrix()
    full = pl.BlockSpec((C, H, D), lambda b: (0, 0, 0))
    pspec = pl.BlockSpec((2 * NT, F), lambda b: (0, 0))
    outb = pl.BlockSpec((OHB * OW, F), lambda b: (b, 0))
    return pl.pallas_call(
        _tc_body,
        grid=(NB,),
        in_specs=[full, full, pspec],
        out_specs=outb,
        out_shape=jax.ShapeDtypeStruct((L, F), jnp.float32),
    )(xe, xo, proj)


OH_SC = 160
NB_TAIL = NB - OH_SC // OHB


def _tc_tail_body(xe, xo, p_ref, prev_ref, o_ref):
    b = pl.program_id(0)
    pm = p_ref[...]
    for k in range(OHB):
        i = jnp.minimum(OH_SC + OHB * b + k, OH - 1)
        o_ref[pl.ds(OW * k, OW), :] = _make_block(xe, xo, pm, i)


def _tc_unfold_tail(image, prev):
    xe, xo = image[0, :, :, 0::2], image[0, :, :, 1::2]
    proj = _proj_matrix()
    full = pl.BlockSpec((C, H, D), lambda b: (0, 0, 0))
    pspec = pl.BlockSpec((2 * NT, F), lambda b: (0, 0))
    hbm = pl.BlockSpec(memory_space=pltpu.MemorySpace.HBM)
    outb = pl.BlockSpec((OHB * OW, F), lambda b: (OH_SC // OHB + b, 0))
    return pl.pallas_call(
        _tc_tail_body,
        grid=(NB_TAIL,),
        in_specs=[full, full, pspec, hbm],
        out_specs=outb,
        out_shape=jax.ShapeDtypeStruct((L, F), jnp.float32),
        input_output_aliases={3: 0},
    )(xe, xo, proj, prev)


NWORK = 32
OH_PER_W = OH_SC // NWORK
CHUNK = 48
SLABS = [(s * CHUNK, min(CHUNK, OW - s * CHUNK))
         for s in range((OW + CHUNK - 1) // CHUNK)]


def _sc_unfold(image):
    mesh = plsc.VectorSubcoreMesh(core_axis_name="c", subcore_axis_name="s")

    @pl.kernel(
        mesh=mesh,
        out_type=jax.ShapeDtypeStruct((L, F), jnp.float32),
        compiler_params=pltpu.CompilerParams(use_tc_tiling_on_sc=False),
        scratch_types=[
            pltpu.VMEM((C, P, W), jnp.float32),
            pltpu.VMEM((CHUNK, F), jnp.float32),
            pltpu.VMEM((CHUNK, F), jnp.float32),
            pltpu.SemaphoreType.DMA,
            pltpu.SemaphoreType.DMA,
            pltpu.SemaphoreType.DMA,
        ],
    )
    def sc_kernel(x_hbm, o_hbm, w_ref, buf0, buf1, sem0, sem1, wsem):
        wid = lax.axis_index("s") * 2 + lax.axis_index("c")
        bufs, sems = [buf0, buf1], [sem0, sem1]

        @pl.loop(0, OH_PER_W)
        def _(j):
            oh = NWORK * j + wid
            pltpu.async_copy(
                x_hbm.at[:, pl.ds(S * oh, P), :], w_ref, wsem).wait()
            pending = [None, None]
            for k, (ow0, n) in enumerate(SLABS):
                b = k % 2
                if pending[b] is not None:
                    pending[b].wait()

                @plsc.parallel_loop(0, n, unroll=2)
                def _(owl, ow0=ow0, b=b):
                    ow = ow0 + owl
                    for t in range(NT):
                        c, kh = divmod(t, P)
                        bufs[b][owl, pl.ds(P * t, P)] = (
                            w_ref[c, kh, pl.ds(S * ow, P)])
                cp = pltpu.make_async_copy(
                    bufs[b].at[pl.ds(0, n), :],
                    o_hbm.at[pl.ds(OW * oh + ow0, n), :],
                    sems[b])
                cp.start()
                pending[b] = cp
            for b in range(2):
                if pending[b] is not None:
                    pending[b].wait()

    return sc_kernel(image[0])


def kernel(input_1, input_2):
    o2_head = _sc_unfold(input_2)
    o1 = _tc_unfold(input_1)
    o2 = _tc_unfold_tail(input_2, o2_head)
    return o1, o2


# speedup vs baseline: 1.3635x; 1.3635x over previous
import numpy as np
import jax
import jax.numpy as jnp
from jax import lax
from jax.experimental import pallas as pl
from jax.experimental.pallas import tpu as pltpu
from jax.experimental.pallas import tpu_sc as plsc

P = 16
S = 2
C = 3
H = W = 512
OH = OW = (H - P) // S + 1
L = OH * OW
F = C * P * P
D = W // 2
NT = C * P


def _proj_matrix():
    p = np.zeros((2 * NT, F), np.float32)
    for t in range(NT):
        for v in range(2):
            for m in range(8):
                p[48 * v + t, 16 * t + 2 * m + v] = 1.0
    return jnp.asarray(p, jnp.bfloat16)


def _window16(ref, i, img_c):
    q = jnp.minimum(i // 4, (H - 24) // 8)
    base = pl.multiple_of(8 * q, 8)
    rows24 = ref[img_c, pl.ds(base, 24), :]
    r = S * i - 8 * q
    return jax.lax.switch(
        r // 2, [lambda k=k: rows24[2 * k:2 * k + P, :] for k in range(5)]
    )


def _rows48(ref, i):
    return jnp.concatenate([_window16(ref, i, c) for c in range(C)], axis=0)


def _make_block(xe, xo, pm, i):
    eo96 = jnp.concatenate([_rows48(xe, i), _rows48(xo, i)], axis=0)
    tr = jnp.swapaxes(eo96, 0, 1)
    hi = tr.astype(jnp.bfloat16)
    lo = (tr - hi.astype(jnp.float32)).astype(jnp.bfloat16)
    dn = (((1,), (0,)), ((), ()))
    rep = (jax.lax.dot_general(hi, pm, dn, preferred_element_type=jnp.float32)
           + jax.lax.dot_general(lo, pm, dn, preferred_element_type=jnp.float32))
    lm = (jax.lax.broadcasted_iota(jnp.int32, (1, F), 1) // 2) % 8
    acc = jnp.where(lm == 0, rep[0:OW, :], 0.0)
    for m in range(1, 8):
        acc = jnp.where(lm == m, rep[m:m + OW, :], acc)
    return acc


OHB = 8
NB = (OH + OHB - 1) // OHB


def _tc_body(xe, xo, p_ref, o_ref):
    b = pl.program_id(0)
    pm = p_ref[...]
    for k in range(OHB):
        i = jnp.minimum(OHB * b + k, OH - 1)
        o_ref[pl.ds(OW * k, OW), :] = _make_block(xe, xo, pm, i)


def _tc_unfold(image):
    xe, xo = image[0, :, :, 0::2], image[0, :, :, 1::2]
    proj = _proj_matrix()
    full = pl.BlockSpec((C, H, D), lambda b: (0, 0, 0))
    pspec = pl.BlockSpec((2 * NT, F), lambda b: (0, 0))
    outb = pl.BlockSpec((OHB * OW, F), lambda b: (b, 0))
    return pl.pallas_call(
        _tc_body,
        grid=(NB,),
        in_specs=[full, full, pspec],
        out_specs=outb,
        out_shape=jax.ShapeDtypeStruct((L, F), jnp.float32),
    )(xe, xo, proj)


NWORK = 32
OH_PER_W = (OH + NWORK - 1) // NWORK
CHUNK = 48
SLABS = [(s * CHUNK, min(CHUNK, OW - s * CHUNK))
         for s in range((OW + CHUNK - 1) // CHUNK)]


def _sc_unfold(image):
    mesh = plsc.VectorSubcoreMesh(core_axis_name="c", subcore_axis_name="s")

    @pl.kernel(
        mesh=mesh,
        out_type=jax.ShapeDtypeStruct((L, F), jnp.float32),
        compiler_params=pltpu.CompilerParams(use_tc_tiling_on_sc=False),
        scratch_types=[
            pltpu.VMEM((C, P, W), jnp.float32),
            pltpu.VMEM((CHUNK, F), jnp.float32),
            pltpu.VMEM((CHUNK, F), jnp.float32),
            pltpu.SemaphoreType.DMA,
            pltpu.SemaphoreType.DMA,
            pltpu.SemaphoreType.DMA,
        ],
    )
    def sc_kernel(x_hbm, o_hbm, w_ref, buf0, buf1, sem0, sem1, wsem):
        wid = lax.axis_index("s") * 2 + lax.axis_index("c")
        bufs, sems = [buf0, buf1], [sem0, sem1]

        @pl.loop(0, OH_PER_W)
        def _(j):
            oh = jnp.minimum(NWORK * j + wid, OH - 1)
            pltpu.async_copy(
                x_hbm.at[:, pl.ds(S * oh, P), :], w_ref, wsem).wait()
            pending = [None, None]
            for k, (ow0, n) in enumerate(SLABS):
                b = k % 2
                if pending[b] is not None:
                    pending[b].wait()

                @plsc.parallel_loop(0, n, unroll=2)
                def _(owl, ow0=ow0, b=b):
                    ow = ow0 + owl
                    for t in range(NT):
                        c, kh = divmod(t, P)
                        bufs[b][owl, pl.ds(P * t, P)] = (
                            w_ref[c, kh, pl.ds(S * ow, P)])
                cp = pltpu.make_async_copy(
                    bufs[b].at[pl.ds(0, n), :],
                    o_hbm.at[pl.ds(OW * oh + ow0, n), :],
                    sems[b])
                cp.start()
                pending[b] = cp
            for b in range(2):
                if pending[b] is not None:
                    pending[b].wait()

    return sc_kernel(image[0])


def kernel(input_1, input_2):
    return _tc_unfold(input_1), _sc_unfold(input_2)
